# baseline (speedup 1.0000x reference)
import jax
import jax.numpy as jnp
from jax.experimental import pallas as pl
from jax.experimental.pallas import tpu as pltpu
from jax.experimental.pallas import tpu_sc as plsc

B, L = 4096, 50
N = B * L
WORD_VOCAB = 1000000
POS_VOCAB = 1000
WORD_DIM = 64
POS_DIM = 32

W = 128
SC_GRID = N // W
TBLK = 4000
OB = 16


def _widen_word(word_table):
    def body(t_ref, o_ref):
        x = t_ref[...]
        o_ref[:, :WORD_DIM] = x
        o_ref[:, WORD_DIM:] = x

    return pl.pallas_call(
        body,
        grid=(WORD_VOCAB // TBLK,),
        in_specs=[pl.BlockSpec((TBLK, WORD_DIM), lambda i: (i, 0))],
        out_specs=pl.BlockSpec((TBLK, 128), lambda i: (i, 0)),
        out_shape=jax.ShapeDtypeStruct((WORD_VOCAB, 128), jnp.float32),
        compiler_params=pltpu.CompilerParams(
            dimension_semantics=("parallel",)),
    )(word_table)


def _widen_small(pos_table, rel_table):
    def body(p_ref, r_ref, po_ref, ro_ref):
        p = p_ref[...]
        r = r_ref[...]
        for g in range(4):
            po_ref[:, g * POS_DIM:(g + 1) * POS_DIM] = p
            ro_ref[:, g * POS_DIM:(g + 1) * POS_DIM] = r

    return pl.pallas_call(
        body,
        out_shape=(
            jax.ShapeDtypeStruct((POS_VOCAB, 128), jnp.float32),
            jax.ShapeDtypeStruct((POS_VOCAB, 128), jnp.float32),
        ),
    )(pos_table, rel_table)


def _sc_gather(wt_wide, pt_wide, rt_wide, widx, pidx, ridx):
    mesh = plsc.VectorSubcoreMesh(core_axis_name="c", subcore_axis_name="s")

    @pl.kernel(
        out_type=(
            jax.ShapeDtypeStruct((N, 128), jnp.float32),
            jax.ShapeDtypeStruct((N, 128), jnp.float32),
            jax.ShapeDtypeStruct((N, 128), jnp.float32),
        ),
        mesh=mesh,
        compiler_params=pltpu.CompilerParams(use_tc_tiling_on_sc=False),
    )
    def kern(wt_hbm, pt_hbm, rt_hbm, wi_hbm, pi_hbm, ri_hbm,
             wo_hbm, po_hbm, ro_hbm):
        def body(wi_v, pi_v, ri_v, wo_v, po_v, ro_v):
            pltpu.sync_copy(wt_hbm.at[wi_v.at[0]], wo_v)
            pltpu.sync_copy(pt_hbm.at[pi_v.at[0]], po_v)
            pltpu.sync_copy(rt_hbm.at[ri_v.at[0]], ro_v)

        pltpu.emit_pipeline(
            body,
            grid=(SC_GRID,),
            in_specs=[
                pl.BlockSpec((1, W), lambda i: (i, 0)),
                pl.BlockSpec((1, W), lambda i: (i, 0)),
                pl.BlockSpec((1, W), lambda i: (i, 0)),
            ],
            out_specs=[
                pl.BlockSpec((W, 128), lambda i: (i, 0)),
                pl.BlockSpec((W, 128), lambda i: (i, 0)),
                pl.BlockSpec((W, 128), lambda i: (i, 0)),
            ],
            core_axis_name=("c", "s"),
            dimension_semantics=(pltpu.PARALLEL,),
        )(wi_hbm, pi_hbm, ri_hbm, wo_hbm, po_hbm, ro_hbm)

    return kern(wt_wide, pt_wide, rt_wide, widx, pidx, ridx)


def _unpack_outputs(word_wide, pos_wide, rel_wide):
    def body(w_ref, p_ref, r_ref, wo_ref, po_ref, ro_ref):
        wo_ref[...] = w_ref[:, :WORD_DIM].reshape(OB, L, WORD_DIM)
        po_ref[...] = p_ref[:, :POS_DIM].reshape(OB, L, POS_DIM)
        ro_ref[...] = r_ref[:, :POS_DIM].reshape(OB, L, POS_DIM)

    return pl.pallas_call(
        body,
        grid=(B // OB,),
        in_specs=[
            pl.BlockSpec((OB * L, 128), lambda i: (i, 0)),
            pl.BlockSpec((OB * L, 128), lambda i: (i, 0)),
            pl.BlockSpec((OB * L, 128), lambda i: (i, 0)),
        ],
        out_specs=[
            pl.BlockSpec((OB, L, WORD_DIM), lambda i: (i, 0, 0)),
            pl.BlockSpec((OB, L, POS_DIM), lambda i: (i, 0, 0)),
            pl.BlockSpec((OB, L, POS_DIM), lambda i: (i, 0, 0)),
        ],
        out_shape=(
            jax.ShapeDtypeStruct((B, L, WORD_DIM), jnp.float32),
            jax.ShapeDtypeStruct((B, L, POS_DIM), jnp.float32),
            jax.ShapeDtypeStruct((B, L, POS_DIM), jnp.float32),
        ),
        compiler_params=pltpu.CompilerParams(
            dimension_semantics=("parallel",)),
    )(word_wide, pos_wide, rel_wide)


@jax.jit
def kernel(word_idxs, pos_idxs, rel_idxs, word_table, pos_table, rel_table):
    wt_wide = _widen_word(word_table)
    pt_wide, rt_wide = _widen_small(pos_table, rel_table)
    widx = word_idxs.reshape(SC_GRID, W)
    pidx = pos_idxs.reshape(SC_GRID, W)
    ridx = rel_idxs.reshape(SC_GRID, W)
    word_wide, pos_wide, rel_wide = _sc_gather(
        wt_wide, pt_wide, rt_wide, widx, pidx, ridx)
    return _unpack_outputs(word_wide, pos_wide, rel_wide)


# speedup vs baseline: 1.2057x; 1.2057x over previous
import jax
import jax.numpy as jnp
from jax.experimental import pallas as pl
from jax.experimental.pallas import tpu as pltpu
from jax.experimental.pallas import tpu_sc as plsc

B, L = 4096, 50
N = B * L
WORD_VOCAB = 1000000
POS_VOCAB = 1000
WORD_DIM = 64
POS_DIM = 32

W = 128
SC_GRID = N // W
TBLK = 4000
OB = 16


def _widen_word(word_table):
    def body(t_ref, o_ref):
        x = t_ref[...]
        o_ref[:, :WORD_DIM] = x
        o_ref[:, WORD_DIM:] = x

    return pl.pallas_call(
        body,
        grid=(WORD_VOCAB // TBLK,),
        in_specs=[pl.BlockSpec((TBLK, WORD_DIM), lambda i: (i, 0))],
        out_specs=pl.BlockSpec((TBLK, 128), lambda i: (i, 0)),
        out_shape=jax.ShapeDtypeStruct((WORD_VOCAB, 128), jnp.float32),
        compiler_params=pltpu.CompilerParams(
            dimension_semantics=("parallel",)),
    )(word_table)


def _widen_small(pos_table, rel_table):
    def body(p_ref, r_ref, po_ref, ro_ref):
        p = p_ref[...]
        r = r_ref[...]
        for g in range(4):
            po_ref[:, g * POS_DIM:(g + 1) * POS_DIM] = p
            ro_ref[:, g * POS_DIM:(g + 1) * POS_DIM] = r

    return pl.pallas_call(
        body,
        out_shape=(
            jax.ShapeDtypeStruct((POS_VOCAB, 128), jnp.float32),
            jax.ShapeDtypeStruct((POS_VOCAB, 128), jnp.float32),
        ),
    )(pos_table, rel_table)


def _sc_gather(wt_wide, pt_wide, rt_wide, widx, pidx, ridx):
    mesh = plsc.VectorSubcoreMesh(core_axis_name="c", subcore_axis_name="s")

    @pl.kernel(
        out_type=(
            jax.ShapeDtypeStruct((N, 128), jnp.float32),
            jax.ShapeDtypeStruct((N, 128), jnp.float32),
            jax.ShapeDtypeStruct((N, 128), jnp.float32),
        ),
        mesh=mesh,
        compiler_params=pltpu.CompilerParams(use_tc_tiling_on_sc=False),
    )
    def kern(wt_hbm, pt_hbm, rt_hbm, wi_hbm, pi_hbm, ri_hbm,
             wo_hbm, po_hbm, ro_hbm):
        def body(wi_v, pi_v, ri_v, wo_v, po_v, ro_v):
            pltpu.sync_copy(wt_hbm.at[wi_v.at[0]], wo_v)
            pltpu.sync_copy(pt_hbm.at[pi_v.at[0]], po_v)
            pltpu.sync_copy(rt_hbm.at[ri_v.at[0]], ro_v)

        pltpu.emit_pipeline(
            body,
            grid=(SC_GRID,),
            in_specs=[
                pl.BlockSpec((1, W), lambda i: (i, 0)),
                pl.BlockSpec((1, W), lambda i: (i, 0)),
                pl.BlockSpec((1, W), lambda i: (i, 0)),
            ],
            out_specs=[
                pl.BlockSpec((W, 128), lambda i: (i, 0)),
                pl.BlockSpec((W, 128), lambda i: (i, 0)),
                pl.BlockSpec((W, 128), lambda i: (i, 0)),
            ],
            core_axis_name=("c", "s"),
            dimension_semantics=(pltpu.PARALLEL,),
        )(wi_hbm, pi_hbm, ri_hbm, wo_hbm, po_hbm, ro_hbm)

    return kern(wt_wide, pt_wide, rt_wide, widx, pidx, ridx)


def _unpack_outputs(word_wide, pos_wide, rel_wide):
    def body(w_ref, p_ref, r_ref, wo_ref, po_ref, ro_ref):
        wo_ref[...] = w_ref[:, :WORD_DIM].reshape(OB, L, WORD_DIM)
        po_ref[...] = p_ref[:, :POS_DIM].reshape(OB, L, POS_DIM)
        ro_ref[...] = r_ref[:, :POS_DIM].reshape(OB, L, POS_DIM)

    return pl.pallas_call(
        body,
        grid=(B // OB,),
        in_specs=[
            pl.BlockSpec((OB * L, 128), lambda i: (i, 0)),
            pl.BlockSpec((OB * L, 128), lambda i: (i, 0)),
            pl.BlockSpec((OB * L, 128), lambda i: (i, 0)),
        ],
        out_specs=[
            pl.BlockSpec((OB, L, WORD_DIM), lambda i: (i, 0, 0)),
            pl.BlockSpec((OB, L, POS_DIM), lambda i: (i, 0, 0)),
            pl.BlockSpec((OB, L, POS_DIM), lambda i: (i, 0, 0)),
        ],
        out_shape=(
            jax.ShapeDtypeStruct((B, L, WORD_DIM), jnp.float32),
            jax.ShapeDtypeStruct((B, L, POS_DIM), jnp.float32),
            jax.ShapeDtypeStruct((B, L, POS_DIM), jnp.float32),
        ),
        compiler_params=pltpu.CompilerParams(
            dimension_semantics=("parallel",)),
    )(word_wide, pos_wide, rel_wide)


@jax.jit
def kernel(word_idxs, pos_idxs, rel_idxs, word_table, pos_table, rel_table):
    wt_wide = jnp.pad(word_table, ((0, 0), (0, 128 - WORD_DIM)))
    pt_wide = jnp.pad(pos_table, ((0, 0), (0, 128 - POS_DIM)))
    rt_wide = jnp.pad(rel_table, ((0, 0), (0, 128 - POS_DIM)))
    widx = word_idxs.reshape(SC_GRID, W)
    pidx = pos_idxs.reshape(SC_GRID, W)
    ridx = rel_idxs.reshape(SC_GRID, W)
    word_wide, pos_wide, rel_wide = _sc_gather(
        wt_wide, pt_wide, rt_wide, widx, pidx, ridx)
    return (word_wide[:, :WORD_DIM].reshape(B, L, WORD_DIM),
            pos_wide[:, :POS_DIM].reshape(B, L, POS_DIM),
            rel_wide[:, :POS_DIM].reshape(B, L, POS_DIM))


# speedup vs baseline: 1.2460x; 1.0334x over previous
import jax
import jax.numpy as jnp
from jax.experimental import pallas as pl
from jax.experimental.pallas import tpu as pltpu
from jax.experimental.pallas import tpu_sc as plsc

B, L = 4096, 50
N = B * L
WORD_VOCAB = 1000000
POS_VOCAB = 1000
WORD_DIM = 64
POS_DIM = 32

W = 128
SC_GRID = N // W
TBLK = 4000
OB = 16


def _widen_word(word_table):
    def body(t_ref, o_ref):
        x = t_ref[...]
        o_ref[:, :WORD_DIM] = x
        o_ref[:, WORD_DIM:] = x

    return pl.pallas_call(
        body,
        grid=(WORD_VOCAB // TBLK,),
        in_specs=[pl.BlockSpec((TBLK, WORD_DIM), lambda i: (i, 0))],
        out_specs=pl.BlockSpec((TBLK, 128), lambda i: (i, 0)),
        out_shape=jax.ShapeDtypeStruct((WORD_VOCAB, 128), jnp.float32),
        compiler_params=pltpu.CompilerParams(
            dimension_semantics=("parallel",)),
    )(word_table)


def _widen_small(pos_table, rel_table):
    def body(p_ref, r_ref, po_ref, ro_ref):
        p = p_ref[...]
        r = r_ref[...]
        for g in range(4):
            po_ref[:, g * POS_DIM:(g + 1) * POS_DIM] = p
            ro_ref[:, g * POS_DIM:(g + 1) * POS_DIM] = r

    return pl.pallas_call(
        body,
        out_shape=(
            jax.ShapeDtypeStruct((POS_VOCAB, 128), jnp.float32),
            jax.ShapeDtypeStruct((POS_VOCAB, 128), jnp.float32),
        ),
    )(pos_table, rel_table)


def _sc_gather_word(wt_wide, widx):
    mesh = plsc.VectorSubcoreMesh(core_axis_name="c", subcore_axis_name="s")

    @pl.kernel(
        out_type=jax.ShapeDtypeStruct((N, 128), jnp.float32),
        mesh=mesh,
        compiler_params=pltpu.CompilerParams(use_tc_tiling_on_sc=False),
    )
    def kern(wt_hbm, wi_hbm, wo_hbm):
        def body(wi_v, wo_v):
            pltpu.sync_copy(wt_hbm.at[wi_v.at[0]], wo_v)

        pltpu.emit_pipeline(
            body,
            grid=(SC_GRID,),
            in_specs=[pl.BlockSpec((1, W), lambda i: (i, 0))],
            out_specs=[pl.BlockSpec((W, 128), lambda i: (i, 0))],
            core_axis_name=("c", "s"),
            dimension_semantics=(pltpu.PARALLEL,),
        )(wi_hbm, wo_hbm)

    return kern(wt_wide, widx)


def _sc_gather_posrel(pt_wide, rt_wide, pidx, ridx):
    mesh = plsc.VectorSubcoreMesh(core_axis_name="c", subcore_axis_name="s")

    @pl.kernel(
        out_type=(
            jax.ShapeDtypeStruct((N, 128), jnp.float32),
            jax.ShapeDtypeStruct((N, 128), jnp.float32),
        ),
        mesh=mesh,
        compiler_params=pltpu.CompilerParams(use_tc_tiling_on_sc=False),
    )
    def kern(pt_hbm, rt_hbm, pi_hbm, ri_hbm, po_hbm, ro_hbm):
        def body(pi_v, ri_v, po_v, ro_v):
            pltpu.sync_copy(pt_hbm.at[pi_v.at[0]], po_v)
            pltpu.sync_copy(rt_hbm.at[ri_v.at[0]], ro_v)

        pltpu.emit_pipeline(
            body,
            grid=(SC_GRID,),
            in_specs=[
                pl.BlockSpec((1, W), lambda i: (i, 0)),
                pl.BlockSpec((1, W), lambda i: (i, 0)),
            ],
            out_specs=[
                pl.BlockSpec((W, 128), lambda i: (i, 0)),
                pl.BlockSpec((W, 128), lambda i: (i, 0)),
            ],
            core_axis_name=("c", "s"),
            dimension_semantics=(pltpu.PARALLEL,),
        )(pi_hbm, ri_hbm, po_hbm, ro_hbm)

    return kern(pt_wide, rt_wide, pidx, ridx)


def _unpack_outputs(word_wide, pos_wide, rel_wide):
    def body(w_ref, p_ref, r_ref, wo_ref, po_ref, ro_ref):
        wo_ref[...] = w_ref[:, :WORD_DIM].reshape(OB, L, WORD_DIM)
        po_ref[...] = p_ref[:, :POS_DIM].reshape(OB, L, POS_DIM)
        ro_ref[...] = r_ref[:, :POS_DIM].reshape(OB, L, POS_DIM)

    return pl.pallas_call(
        body,
        grid=(B // OB,),
        in_specs=[
            pl.BlockSpec((OB * L, 128), lambda i: (i, 0)),
            pl.BlockSpec((OB * L, 128), lambda i: (i, 0)),
            pl.BlockSpec((OB * L, 128), lambda i: (i, 0)),
        ],
        out_specs=[
            pl.BlockSpec((OB, L, WORD_DIM), lambda i: (i, 0, 0)),
            pl.BlockSpec((OB, L, POS_DIM), lambda i: (i, 0, 0)),
            pl.BlockSpec((OB, L, POS_DIM), lambda i: (i, 0, 0)),
        ],
        out_shape=(
            jax.ShapeDtypeStruct((B, L, WORD_DIM), jnp.float32),
            jax.ShapeDtypeStruct((B, L, POS_DIM), jnp.float32),
            jax.ShapeDtypeStruct((B, L, POS_DIM), jnp.float32),
        ),
        compiler_params=pltpu.CompilerParams(
            dimension_semantics=("parallel",)),
    )(word_wide, pos_wide, rel_wide)


@jax.jit
def kernel(word_idxs, pos_idxs, rel_idxs, word_table, pos_table, rel_table):
    pt_wide = jnp.pad(pos_table, ((0, 0), (0, 128 - POS_DIM)))
    rt_wide = jnp.pad(rel_table, ((0, 0), (0, 128 - POS_DIM)))
    pidx = pos_idxs.reshape(SC_GRID, W)
    ridx = rel_idxs.reshape(SC_GRID, W)
    pos_wide, rel_wide = _sc_gather_posrel(pt_wide, rt_wide, pidx, ridx)

    wt_wide = jnp.pad(word_table, ((0, 0), (0, 128 - WORD_DIM)))
    widx = word_idxs.reshape(SC_GRID, W)
    word_wide = _sc_gather_word(wt_wide, widx)

    return (word_wide[:, :WORD_DIM].reshape(B, L, WORD_DIM),
            pos_wide[:, :POS_DIM].reshape(B, L, POS_DIM),
            rel_wide[:, :POS_DIM].reshape(B, L, POS_DIM))


# speedup vs baseline: 1.2750x; 1.0233x over previous
import jax
import jax.numpy as jnp
from jax.experimental import pallas as pl
from jax.experimental.pallas import tpu as pltpu
from jax.experimental.pallas import tpu_sc as plsc

B, L = 4096, 50
N = B * L
WORD_VOCAB = 1000000
POS_VOCAB = 1000
WORD_DIM = 64
POS_DIM = 32

W = 128
SC_GRID = N // W
TBLK = 4000
OB = 16


def _widen_word(word_table):
    def body(t_ref, o_ref):
        x = t_ref[...]
        o_ref[:, :WORD_DIM] = x
        o_ref[:, WORD_DIM:] = x

    return pl.pallas_call(
        body,
        grid=(WORD_VOCAB // TBLK,),
        in_specs=[pl.BlockSpec((TBLK, WORD_DIM), lambda i: (i, 0))],
        out_specs=pl.BlockSpec((TBLK, 128), lambda i: (i, 0)),
        out_shape=jax.ShapeDtypeStruct((WORD_VOCAB, 128), jnp.float32),
        compiler_params=pltpu.CompilerParams(
            dimension_semantics=("parallel",)),
    )(word_table)


def _widen_small(pos_table, rel_table):
    def body(p_ref, r_ref, po_ref, ro_ref):
        p = p_ref[...]
        r = r_ref[...]
        for g in range(4):
            po_ref[:, g * POS_DIM:(g + 1) * POS_DIM] = p
            ro_ref[:, g * POS_DIM:(g + 1) * POS_DIM] = r

    return pl.pallas_call(
        body,
        out_shape=(
            jax.ShapeDtypeStruct((POS_VOCAB, 128), jnp.float32),
            jax.ShapeDtypeStruct((POS_VOCAB, 128), jnp.float32),
        ),
    )(pos_table, rel_table)


def _sc_gather_word(wt_wide, widx):
    mesh = plsc.VectorSubcoreMesh(core_axis_name="c", subcore_axis_name="s")

    @pl.kernel(
        out_type=jax.ShapeDtypeStruct((N, 128), jnp.float32),
        mesh=mesh,
        compiler_params=pltpu.CompilerParams(use_tc_tiling_on_sc=False),
    )
    def kern(wt_hbm, wi_hbm, wo_hbm):
        def body(wi_v, wo_v):
            pltpu.sync_copy(wt_hbm.at[wi_v.at[0]], wo_v)

        pltpu.emit_pipeline(
            body,
            grid=(SC_GRID,),
            in_specs=[pl.BlockSpec((1, W), lambda i: (i, 0))],
            out_specs=[pl.BlockSpec((W, 128), lambda i: (i, 0))],
            core_axis_name=("c", "s"),
            dimension_semantics=(pltpu.PARALLEL,),
        )(wi_hbm, wo_hbm)

    return kern(wt_wide, widx)


def _sc_gather_posrel(pt_wide, rt_wide, pidx, ridx):
    mesh = plsc.VectorSubcoreMesh(core_axis_name="c", subcore_axis_name="s")

    @pl.kernel(
        out_type=(
            jax.ShapeDtypeStruct((N, POS_DIM), jnp.float32),
            jax.ShapeDtypeStruct((N, POS_DIM), jnp.float32),
        ),
        mesh=mesh,
        compiler_params=pltpu.CompilerParams(use_tc_tiling_on_sc=False),
    )
    def kern(pt_hbm, rt_hbm, pi_hbm, ri_hbm, po_hbm, ro_hbm):
        def body(pi_v, ri_v, po_v, ro_v):
            pltpu.sync_copy(pt_hbm.at[pi_v.at[0]], po_v)
            pltpu.sync_copy(rt_hbm.at[ri_v.at[0]], ro_v)

        pltpu.emit_pipeline(
            body,
            grid=(SC_GRID,),
            in_specs=[
                pl.BlockSpec((1, W), lambda i: (i, 0)),
                pl.BlockSpec((1, W), lambda i: (i, 0)),
            ],
            out_specs=[
                pl.BlockSpec((W, POS_DIM), lambda i: (i, 0)),
                pl.BlockSpec((W, POS_DIM), lambda i: (i, 0)),
            ],
            core_axis_name=("c", "s"),
            dimension_semantics=(pltpu.PARALLEL,),
        )(pi_hbm, ri_hbm, po_hbm, ro_hbm)

    return kern(pt_wide, rt_wide, pidx, ridx)


def _unpack_outputs(word_wide, pos_wide, rel_wide):
    def body(w_ref, p_ref, r_ref, wo_ref, po_ref, ro_ref):
        wo_ref[...] = w_ref[:, :WORD_DIM].reshape(OB, L, WORD_DIM)
        po_ref[...] = p_ref[:, :POS_DIM].reshape(OB, L, POS_DIM)
        ro_ref[...] = r_ref[:, :POS_DIM].reshape(OB, L, POS_DIM)

    return pl.pallas_call(
        body,
        grid=(B // OB,),
        in_specs=[
            pl.BlockSpec((OB * L, 128), lambda i: (i, 0)),
            pl.BlockSpec((OB * L, 128), lambda i: (i, 0)),
            pl.BlockSpec((OB * L, 128), lambda i: (i, 0)),
        ],
        out_specs=[
            pl.BlockSpec((OB, L, WORD_DIM), lambda i: (i, 0, 0)),
            pl.BlockSpec((OB, L, POS_DIM), lambda i: (i, 0, 0)),
            pl.BlockSpec((OB, L, POS_DIM), lambda i: (i, 0, 0)),
        ],
        out_shape=(
            jax.ShapeDtypeStruct((B, L, WORD_DIM), jnp.float32),
            jax.ShapeDtypeStruct((B, L, POS_DIM), jnp.float32),
            jax.ShapeDtypeStruct((B, L, POS_DIM), jnp.float32),
        ),
        compiler_params=pltpu.CompilerParams(
            dimension_semantics=("parallel",)),
    )(word_wide, pos_wide, rel_wide)


@jax.jit
def kernel(word_idxs, pos_idxs, rel_idxs, word_table, pos_table, rel_table):
    pidx = pos_idxs.reshape(SC_GRID, W)
    ridx = rel_idxs.reshape(SC_GRID, W)
    pos_out, rel_out = _sc_gather_posrel(pos_table, rel_table, pidx, ridx)

    wt_wide = jnp.pad(word_table, ((0, 0), (0, 128 - WORD_DIM)))
    widx = word_idxs.reshape(SC_GRID, W)
    word_wide = _sc_gather_word(wt_wide, widx)

    return (word_wide[:, :WORD_DIM].reshape(B, L, WORD_DIM),
            pos_out.reshape(B, L, POS_DIM),
            rel_out.reshape(B, L, POS_DIM))
